# baseline (speedup 1.0000x reference)
import functools

import jax
import jax.numpy as jnp
from jax import lax
from jax.experimental import pallas as pl
from jax.experimental.pallas import tpu as pltpu
from jax.experimental.pallas import tpu_sc as plsc

T_OUT = 2048


@functools.cache
def _make_unpack(N, D, B):
    info = plsc.get_sparse_core_info()
    NC, NS, L = info.num_cores, info.num_subcores, info.num_lanes
    NW = NC * NS
    G = 32
    NGT = (B * T_OUT) // G
    KG = NGT // NW
    assert KG * NW == NGT and T_OUT % G == 0
    GB = T_OUT // G
    NBUF = 3
    ZR = 16

    mesh = plsc.VectorSubcoreMesh(core_axis_name="c", subcore_axis_name="s")

    @functools.partial(
        pl.kernel,
        mesh=mesh,
        out_type=jax.ShapeDtypeStruct((B * T_OUT, D), jnp.float32),
        scratch_types=[
            pltpu.VMEM((KG * G,), jnp.int32),
            pltpu.VMEM((L,), jnp.int32),
            *[pltpu.VMEM((G, D), jnp.float32) for _ in range(NBUF)],
            pltpu.VMEM((ZR, D), jnp.float32),
            *[pltpu.SemaphoreType.DMA for _ in range(2 * NBUF)],
        ],
    )
    def unpack(data_hbm, len_hbm, out_hbm, idx_v, len_v, *rest):
        bufs = rest[:NBUF]
        zbuf = rest[NBUF]
        gsems = rest[NBUF + 1:2 * NBUF + 1]
        wsems = rest[2 * NBUF + 1:3 * NBUF + 1]
        wid = lax.axis_index("s") * NC + lax.axis_index("c")

        len_v[...] = jnp.zeros((L,), jnp.int32)
        len_cp = pltpu.make_async_copy(len_hbm, len_v.at[pl.ds(0, B)],
                                       gsems[0])
        len_cp.start()

        def zrow(i, carry):
            def zcol(c, carry2):
                zbuf[i, pl.ds(c * L, L)] = jnp.zeros((L,), jnp.float32)
                return carry2

            return lax.fori_loop(0, D // L, zcol, carry)

        lax.fori_loop(0, ZR, zrow, 0)
        len_cp.wait()
        lanes = lax.iota(jnp.int32, L)
        lv = len_v[...]
        lens = [lv[j] for j in range(B)]

        def meta(k):
            gg = wid + k * NW
            bk = gg // GB
            t0k = (gg % GB) * G
            lb = lens[0] * 0
            for j in range(B):
                lb = jnp.where(bk == j, lens[j], lb)
            vk = jnp.clip(lb - t0k, 0, G)
            return gg, bk, t0k, vk

        def idx_fill(k, bk, t0k):
            for s in range(G // L):
                t_vec = t0k + s * L + lanes
                acc = jnp.zeros((L,), jnp.int32)
                for lj in lens:
                    acc = acc + jnp.minimum(t_vec, lj)
                idx_v[pl.ds(k * G + s * L, L)] = jnp.minimum(
                    acc + bk, N - 1)

        def gather(k, p):
            return pltpu.make_async_copy(
                data_hbm.at[idx_v.at[pl.ds(k * G, G)]], bufs[p], gsems[p]
            )

        def write(gg, p, src):
            return pltpu.make_async_copy(
                src, out_hbm.at[pl.ds(gg * G, G)], wsems[p]
            )

        for k in range(min(NBUF, KG)):
            gg, bk, t0k, vk = meta(k)
            idx_fill(k, bk, t0k)

            @pl.when(vk > 0)
            def _(k=k, p=k % NBUF):
                gather(k, p).start()

        def idx_body(k, carry):
            _, bk, t0k, _ = meta(k)
            idx_fill(k, bk, t0k)
            return carry

        lax.fori_loop(min(NBUF, KG), KG, idx_body, 0)

        def slot(k, p):
            gg, bk, t0k, vk = meta(k)

            @pl.when(vk > 0)
            def _():
                gather(k, p).wait()

                @pl.when(vk < G)
                def _():
                    def zfix(i, carry):
                        def zcol(c, carry2):
                            bufs[p][i, pl.ds(c * L, L)] = jnp.zeros(
                                (L,), jnp.float32)
                            return carry2

                        return lax.fori_loop(0, D // L, zcol, carry)

                    lax.fori_loop(vk, G, zfix, 0)

                write(gg, p, bufs[p]).start()

            @pl.when(vk <= 0)
            def _():
                for q in range(G // ZR):
                    pltpu.make_async_copy(
                        zbuf,
                        out_hbm.at[pl.ds(gg * G + q * ZR, ZR)],
                        wsems[p],
                    ).start()

            @pl.when(k + NBUF < KG)
            def _():
                vn = meta(k + NBUF)[3]
                write(gg, p, bufs[p]).wait()

                @pl.when(vn > 0)
                def _():
                    gather(k + NBUF, p).start()

        def main_body(j, carry):
            for i in range(NBUF):
                k = j * NBUF + i

                @pl.when(k < KG)
                def _(k=k, i=i):
                    slot(k, i)

            return carry

        lax.fori_loop(0, (KG + NBUF - 1) // NBUF, main_body, 0)

        for k in range(max(0, KG - NBUF), KG):
            write(wid + k * NW, k % NBUF, bufs[k % NBUF]).wait()

    return unpack


def kernel(data, lengths):
    N, D = data.shape
    B = lengths.shape[0]
    out = _make_unpack(N, D, B)(data, lengths.astype(jnp.int32))
    return out.reshape(B, T_OUT, D), lengths


# speedup vs baseline: 1.0541x; 1.0541x over previous
import functools

import jax
import jax.numpy as jnp
from jax import lax
from jax.experimental import pallas as pl
from jax.experimental.pallas import tpu as pltpu
from jax.experimental.pallas import tpu_sc as plsc

T_OUT = 2048


@functools.cache
def _make_unpack(N, D, B):
    info = plsc.get_sparse_core_info()
    NC, NS, L = info.num_cores, info.num_subcores, info.num_lanes
    NW = NC * NS
    G = 32
    NGT = (B * T_OUT) // G
    KG = NGT // NW
    assert KG * NW == NGT and T_OUT % G == 0
    GB = T_OUT // G
    NBUF = 3
    ZR = 16

    mesh = plsc.VectorSubcoreMesh(core_axis_name="c", subcore_axis_name="s")

    @functools.partial(
        pl.kernel,
        mesh=mesh,
        out_type=jax.ShapeDtypeStruct((B * T_OUT, D), jnp.float32),
        scratch_types=[
            pltpu.VMEM((KG * G,), jnp.int32),
            pltpu.VMEM((L,), jnp.int32),
            *[pltpu.VMEM((G, D), jnp.float32) for _ in range(NBUF)],
            pltpu.VMEM((ZR, D), jnp.float32),
            *[pltpu.SemaphoreType.DMA for _ in range(2 * NBUF)],
        ],
    )
    def unpack(data_hbm, len_hbm, out_hbm, idx_v, len_v, *rest):
        bufs = rest[:NBUF]
        zbuf = rest[NBUF]
        gsems = rest[NBUF + 1:2 * NBUF + 1]
        wsems = rest[2 * NBUF + 1:3 * NBUF + 1]
        wid = lax.axis_index("s") * NC + lax.axis_index("c")

        len_v[...] = jnp.zeros((L,), jnp.int32)
        pltpu.sync_copy(len_hbm, len_v.at[pl.ds(0, B)])
        lanes = lax.iota(jnp.int32, L)
        lv = len_v[...]
        lens = [lv[j] for j in range(B)]

        def meta(k):
            gg = wid + k * NW
            bk = gg // GB
            t0k = (gg % GB) * G
            lb = lens[0] * 0
            for j in range(B):
                lb = jnp.where(bk == j, lens[j], lb)
            vk = jnp.clip(lb - t0k, 0, G)
            return gg, bk, t0k, vk

        def idx_fill(k, bk, t0k):
            for s in range(G // L):
                t_vec = t0k + s * L + lanes
                acc = jnp.zeros((L,), jnp.int32)
                for lj in lens:
                    acc = acc + jnp.minimum(t_vec, lj)
                idx_v[pl.ds(k * G + s * L, L)] = jnp.minimum(
                    acc + bk, N - 1)

        def gather(k, p):
            return pltpu.make_async_copy(
                data_hbm.at[idx_v.at[pl.ds(k * G, G)]], bufs[p], gsems[p]
            )

        def write(gg, p, src):
            return pltpu.make_async_copy(
                src, out_hbm.at[pl.ds(gg * G, G)], wsems[p]
            )

        for k in range(min(NBUF, KG)):
            gg, bk, t0k, vk = meta(k)
            idx_fill(k, bk, t0k)

            @pl.when(vk > 0)
            def _(k=k, p=k % NBUF):
                gather(k, p).start()

        def zrow(i, carry):
            def zcol(c, carry2):
                zbuf[i, pl.ds(c * L, L)] = jnp.zeros((L,), jnp.float32)
                return carry2

            return lax.fori_loop(0, D // L, zcol, carry)

        lax.fori_loop(0, ZR, zrow, 0)

        def idx_body(k, carry):
            _, bk, t0k, _ = meta(k)
            idx_fill(k, bk, t0k)
            return carry

        lax.fori_loop(min(NBUF, KG), KG, idx_body, 0)

        def slot(k, p):
            gg, bk, t0k, vk = meta(k)

            @pl.when(vk > 0)
            def _():
                gather(k, p).wait()

                @pl.when(vk < G)
                def _():
                    def zfix(i, carry):
                        def zcol(c, carry2):
                            bufs[p][i, pl.ds(c * L, L)] = jnp.zeros(
                                (L,), jnp.float32)
                            return carry2

                        return lax.fori_loop(0, D // L, zcol, carry)

                    lax.fori_loop(vk, G, zfix, 0)

                write(gg, p, bufs[p]).start()

            @pl.when(vk <= 0)
            def _():
                for q in range(G // ZR):
                    pltpu.make_async_copy(
                        zbuf,
                        out_hbm.at[pl.ds(gg * G + q * ZR, ZR)],
                        wsems[p],
                    ).start()

            @pl.when(k + NBUF < KG)
            def _():
                vn = meta(k + NBUF)[3]
                write(gg, p, bufs[p]).wait()

                @pl.when(vn > 0)
                def _():
                    gather(k + NBUF, p).start()

        def main_body(j, carry):
            for i in range(NBUF):
                k = j * NBUF + i

                @pl.when(k < KG)
                def _(k=k, i=i):
                    slot(k, i)

            return carry

        lax.fori_loop(0, (KG + NBUF - 1) // NBUF, main_body, 0)

        for k in range(max(0, KG - NBUF), KG):
            write(wid + k * NW, k % NBUF, bufs[k % NBUF]).wait()

    return unpack


def kernel(data, lengths):
    N, D = data.shape
    B = lengths.shape[0]
    out = _make_unpack(N, D, B)(data, lengths.astype(jnp.int32))
    return out.reshape(B, T_OUT, D), lengths
